# baseline (speedup 1.0000x reference)
import functools

import jax
import jax.numpy as jnp
from jax import lax
from jax.experimental import pallas as pl
from jax.experimental.pallas import tpu as pltpu
from jax.experimental.pallas import tpu_sc as plsc

_L = 16
_NSUB = 16
_NCORE = 2
_BINS = 1024
_CH = 8192


def _sc_body(n, o_hbm, t_hbm, s_out, meta_out,
             o_chunk, t_chunk, e_buf, hist, redhist, rowbuf,
             stage_f, meta_stage, shared):
    c = lax.axis_index("c")
    s = lax.axis_index("s")
    per_sub = n // _NSUB
    k = int(n * 0.97)
    nvec_ch = _CH // _L
    lanes = lax.iota(jnp.int32, _L)
    ones = jnp.ones((_L,), jnp.int32)

    def zero_hist():
        def zb(i, _):
            hist[pl.ds(i * _L, _L)] = jnp.zeros((_L,), jnp.int32)
            return 0
        lax.fori_loop(0, (_NSUB * _BINS) // _L, zb, 0)

    def lane_reduce():
        def rb(v, _):
            def rrow(l, acc):
                return acc + hist[pl.ds(l * _BINS + v * _L, _L)]
            acc = lax.fori_loop(0, _NSUB, rrow, jnp.zeros((_L,), jnp.int32))
            redhist[pl.ds(v * _L, _L)] = acc
            return 0
        lax.fori_loop(0, _BINS // _L, rb, 0)

    def combine_across_tiles():
        pltpu.sync_copy(redhist, shared.at[s])
        plsc.subcore_barrier()
        pltpu.sync_copy(shared.at[0], redhist)
        for r in range(1, _NSUB):
            pltpu.sync_copy(shared.at[r], rowbuf)
            def ab(v, _):
                redhist[pl.ds(v * _L, _L)] = (
                    redhist[pl.ds(v * _L, _L)] + rowbuf[pl.ds(v * _L, _L)])
                return 0
            lax.fori_loop(0, _BINS // _L, ab, 0)
        plsc.subcore_barrier()

    def scan_for_crossing(start_count):
        def sb(i, carry):
            running, found, bsel, below = carry
            v = redhist[pl.ds(i * _L, _L)]
            ssum = jnp.sum(v)
            cum = plsc.cumsum(v)
            within = (running + cum) >= k
            lane_pos = jnp.sum(jnp.where(within, 0, 1))
            below_here = running + jnp.sum(jnp.where(lanes < lane_pos, v, 0))
            crossed = jnp.logical_and(running + ssum >= k, found == 0)
            bsel = jnp.where(crossed, i * _L + lane_pos, bsel)
            below = jnp.where(crossed, below_here, below)
            found = jnp.where(crossed, jnp.int32(1), found)
            return running + ssum, found, bsel, below
        init = (start_count, jnp.int32(0), jnp.int32(0), jnp.int32(0))
        _, _, bsel, below = lax.fori_loop(0, _BINS // _L, sb, init)
        return bsel, below

    zero_hist()
    base = s * per_sub
    for ch in range(per_sub // _CH):
        off = base + ch * _CH
        pltpu.sync_copy(o_hbm.at[pl.ds(off, _CH)], o_chunk)
        pltpu.sync_copy(t_hbm.at[pl.ds(off, _CH)], t_chunk)

        def pa(i, _, ch=ch):
            o = o_chunk[pl.ds(i * _L, _L)]
            t = t_chunk[pl.ds(i * _L, _L)]
            r = (t - o) / t
            e = r * r
            e_buf[pl.ds(ch * _CH + i * _L, _L)] = e
            bits = lax.bitcast_convert_type(e, jnp.int32)
            b1 = lax.shift_right_logical(bits, 21)
            plsc.addupdate_scatter(hist, [lanes * _BINS + b1], ones)
            return 0
        lax.fori_loop(0, nvec_ch, pa, 0)

    lane_reduce()
    combine_across_tiles()
    b1_sel, below1 = scan_for_crossing(jnp.int32(0))

    zero_hist()

    def pb(i, _):
        e = e_buf[pl.ds(i * _L, _L)]
        bits = lax.bitcast_convert_type(e, jnp.int32)
        eq = lax.shift_right_logical(bits, 21) == b1_sel
        b2 = lax.bitwise_and(lax.shift_right_logical(bits, 11),
                             jnp.int32(_BINS - 1))
        plsc.addupdate_scatter(hist, [lanes * _BINS + b2], ones, mask=eq)
        return 0
    lax.fori_loop(0, per_sub // _L, pb, 0)

    lane_reduce()
    combine_across_tiles()
    b2_sel, below2 = scan_for_crossing(below1)

    t_bits = lax.shift_left(b1_sel, 21) | lax.shift_left(b2_sel, 11)

    half = per_sub // _NCORE
    start = c * half

    def pc(i, acc):
        e = e_buf[pl.ds(start + i * _L, _L)]
        bits = lax.bitcast_convert_type(e, jnp.int32)
        return acc + jnp.where(bits < t_bits, e, jnp.float32(0.0))
    sacc = lax.fori_loop(0, half // _L, pc, jnp.zeros((_L,), jnp.float32))

    stage_f[...] = sacc
    wid = s * _NCORE + c
    pltpu.sync_copy(stage_f, s_out.at[pl.ds(wid * _L, _L)])

    @pl.when(jnp.logical_and(c == 0, s == 0))
    def _():
        meta_stage[...] = jnp.where(lanes == 0, t_bits, below2)
        pltpu.sync_copy(meta_stage, meta_out)


def kernel(output, target):
    n = output.shape[0]
    k = int(n * 0.97)
    mesh = plsc.VectorSubcoreMesh(
        core_axis_name="c", subcore_axis_name="s",
        num_cores=_NCORE, num_subcores=_NSUB)
    call = pl.kernel(
        functools.partial(_sc_body, n),
        out_type=(
            jax.ShapeDtypeStruct((_NCORE * _NSUB * _L,), jnp.float32),
            jax.ShapeDtypeStruct((_L,), jnp.int32),
        ),
        mesh=mesh,
        compiler_params=pltpu.CompilerParams(needs_layout_passes=False),
        scratch_types=[
            pltpu.VMEM((_CH,), jnp.float32),
            pltpu.VMEM((_CH,), jnp.float32),
            pltpu.VMEM((n // _NSUB,), jnp.float32),
            pltpu.VMEM((_NSUB * _BINS,), jnp.int32),
            pltpu.VMEM((_BINS,), jnp.int32),
            pltpu.VMEM((_BINS,), jnp.int32),
            pltpu.VMEM((_L,), jnp.float32),
            pltpu.VMEM((_L,), jnp.int32),
            pltpu.VMEM_SHARED((_NSUB, _BINS), jnp.int32),
        ],
    )
    s_parts, meta = call(output, target)
    t_val = lax.bitcast_convert_type(meta[0], jnp.float32)
    c_below = meta[1]
    s_total = jnp.sum(s_parts)
    return (s_total + (k - c_below).astype(jnp.float32) * t_val) / jnp.float32(k)


# speedup vs baseline: 1.1551x; 1.1551x over previous
import functools

import jax
import jax.numpy as jnp
from jax import lax
from jax.experimental import pallas as pl
from jax.experimental.pallas import tpu as pltpu
from jax.experimental.pallas import tpu_sc as plsc

_L = 16
_NSUB = 16
_NCORE = 2
_BINS = 1024
_CH = 8192
_U = 8


def _sc_body(n, o_hbm, t_hbm, s_out, meta_out,
             o_chunk, t_chunk, e_buf, hist, redhist, rowbuf,
             stage_f, meta_stage, shared):
    c = lax.axis_index("c")
    s = lax.axis_index("s")
    per_sub = n // _NSUB
    k = int(n * 0.97)
    nvec_ch = _CH // _L
    lanes = lax.iota(jnp.int32, _L)
    ones = jnp.ones((_L,), jnp.int32)

    def zero_hist():
        zv = jnp.zeros((_L,), jnp.int32)

        def zb(i, _):
            for u in range(_U):
                hist[pl.ds((i * _U + u) * _L, _L)] = zv
            return 0
        lax.fori_loop(0, (_NSUB * _BINS) // (_L * _U), zb, 0)

    def lane_reduce():
        def rb(v, _):
            acc = hist[pl.ds(v * _L, _L)]
            for l in range(1, _NSUB):
                acc = acc + hist[pl.ds(l * _BINS + v * _L, _L)]
            redhist[pl.ds(v * _L, _L)] = acc
            return 0
        lax.fori_loop(0, _BINS // _L, rb, 0)

    def combine_across_tiles():
        pltpu.sync_copy(redhist, shared.at[s])
        plsc.subcore_barrier()
        pltpu.sync_copy(shared.at[0], redhist)
        for r in range(1, _NSUB):
            pltpu.sync_copy(shared.at[r], rowbuf)
            def ab(v, _):
                for u in range(4):
                    o = (v * 4 + u) * _L
                    redhist[pl.ds(o, _L)] = (
                        redhist[pl.ds(o, _L)] + rowbuf[pl.ds(o, _L)])
                return 0
            lax.fori_loop(0, _BINS // (_L * 4), ab, 0)
        plsc.subcore_barrier()

    def scan_for_crossing(start_count):
        def sb(i, carry):
            running, found, bsel, below = carry
            v = redhist[pl.ds(i * _L, _L)]
            ssum = jnp.sum(v)
            cum = plsc.cumsum(v)
            within = (running + cum) >= k
            lane_pos = jnp.sum(jnp.where(within, 0, 1))
            below_here = running + jnp.sum(jnp.where(lanes < lane_pos, v, 0))
            crossed = jnp.logical_and(running + ssum >= k, found == 0)
            bsel = jnp.where(crossed, i * _L + lane_pos, bsel)
            below = jnp.where(crossed, below_here, below)
            found = jnp.where(crossed, jnp.int32(1), found)
            return running + ssum, found, bsel, below
        init = (start_count, jnp.int32(0), jnp.int32(0), jnp.int32(0))
        _, _, bsel, below = lax.fori_loop(0, _BINS // _L, sb, init)
        return bsel, below

    zero_hist()
    base = s * per_sub
    for ch in range(per_sub // _CH):
        off = base + ch * _CH
        pltpu.sync_copy(o_hbm.at[pl.ds(off, _CH)], o_chunk)
        pltpu.sync_copy(t_hbm.at[pl.ds(off, _CH)], t_chunk)

        def pa(i, _, ch=ch):
            for u in range(_U):
                off_v = (i * _U + u) * _L
                o = o_chunk[pl.ds(off_v, _L)]
                t = t_chunk[pl.ds(off_v, _L)]
                r = (t - o) / t
                e = r * r
                e_buf[pl.ds(ch * _CH + off_v, _L)] = e
                bits = lax.bitcast_convert_type(e, jnp.int32)
                b1 = lax.shift_right_logical(bits, 21)
                plsc.addupdate_scatter(hist, [lanes * _BINS + b1], ones)
            return 0
        lax.fori_loop(0, nvec_ch // _U, pa, 0)

    lane_reduce()
    combine_across_tiles()
    b1_sel, below1 = scan_for_crossing(jnp.int32(0))

    zero_hist()

    def pb(i, _):
        for u in range(_U):
            off_v = (i * _U + u) * _L
            e = e_buf[pl.ds(off_v, _L)]
            bits = lax.bitcast_convert_type(e, jnp.int32)
            eq = lax.shift_right_logical(bits, 21) == b1_sel
            b2 = lax.bitwise_and(lax.shift_right_logical(bits, 11),
                                 jnp.int32(_BINS - 1))
            plsc.addupdate_scatter(hist, [lanes * _BINS + b2], ones, mask=eq)
        return 0
    lax.fori_loop(0, per_sub // (_L * _U), pb, 0)

    lane_reduce()
    combine_across_tiles()
    b2_sel, below2 = scan_for_crossing(below1)

    t_bits = lax.shift_left(b1_sel, 21) | lax.shift_left(b2_sel, 11)

    half = per_sub // _NCORE
    start = c * half

    def pc(i, accs):
        a0, a1 = accs
        for u in range(_U):
            off_v = start + (i * _U + u) * _L
            e = e_buf[pl.ds(off_v, _L)]
            bits = lax.bitcast_convert_type(e, jnp.int32)
            contrib = jnp.where(bits < t_bits, e, jnp.float32(0.0))
            if u % 2 == 0:
                a0 = a0 + contrib
            else:
                a1 = a1 + contrib
        return a0, a1
    z = jnp.zeros((_L,), jnp.float32)
    a0, a1 = lax.fori_loop(0, half // (_L * _U), pc, (z, z))
    sacc = a0 + a1

    stage_f[...] = sacc
    wid = s * _NCORE + c
    pltpu.sync_copy(stage_f, s_out.at[pl.ds(wid * _L, _L)])

    @pl.when(jnp.logical_and(c == 0, s == 0))
    def _():
        meta_stage[...] = jnp.where(lanes == 0, t_bits, below2)
        pltpu.sync_copy(meta_stage, meta_out)


def kernel(output, target):
    n = output.shape[0]
    k = int(n * 0.97)
    mesh = plsc.VectorSubcoreMesh(
        core_axis_name="c", subcore_axis_name="s",
        num_cores=_NCORE, num_subcores=_NSUB)
    call = pl.kernel(
        functools.partial(_sc_body, n),
        out_type=(
            jax.ShapeDtypeStruct((_NCORE * _NSUB * _L,), jnp.float32),
            jax.ShapeDtypeStruct((_L,), jnp.int32),
        ),
        mesh=mesh,
        compiler_params=pltpu.CompilerParams(needs_layout_passes=False),
        scratch_types=[
            pltpu.VMEM((_CH,), jnp.float32),
            pltpu.VMEM((_CH,), jnp.float32),
            pltpu.VMEM((n // _NSUB,), jnp.float32),
            pltpu.VMEM((_NSUB * _BINS,), jnp.int32),
            pltpu.VMEM((_BINS,), jnp.int32),
            pltpu.VMEM((_BINS,), jnp.int32),
            pltpu.VMEM((_L,), jnp.float32),
            pltpu.VMEM((_L,), jnp.int32),
            pltpu.VMEM_SHARED((_NSUB, _BINS), jnp.int32),
        ],
    )
    s_parts, meta = call(output, target)
    t_val = lax.bitcast_convert_type(meta[0], jnp.float32)
    c_below = meta[1]
    s_total = jnp.sum(s_parts)
    return (s_total + (k - c_below).astype(jnp.float32) * t_val) / jnp.float32(k)


# speedup vs baseline: 1.3272x; 1.1489x over previous
import functools

import jax
import jax.numpy as jnp
from jax import lax
from jax.experimental import pallas as pl
from jax.experimental.pallas import tpu as pltpu
from jax.experimental.pallas import tpu_sc as plsc

_L = 16
_NSUB = 16
_NCORE = 2
_BINS = 1024
_CH = 8192
_U = 8
_SLICE = _BINS // _NSUB


def _sc_body(n, o_hbm, t_hbm, out_hbm,
             tb0, tb1, e_buf, hist, slicebuf, redslice, totstage, totbuf,
             stage_f, fbuf, shared_big, shared_tot, shared_f,
             sem_o, sem_t0, sem_t1):
    c = lax.axis_index("c")
    s = lax.axis_index("s")
    per_sub = n // _NSUB
    k = int(n * 0.97)
    lanes = lax.iota(jnp.int32, _L)
    ones = jnp.ones((_L,), jnp.int32)
    zv = jnp.zeros((_L,), jnp.int32)

    def zero_hist():
        def zb(i, _):
            for u in range(_U):
                hist[pl.ds((i * _U + u) * _L, _L)] = zv
            return 0
        lax.fori_loop(0, (_BINS * _L) // (_L * _U), zb, 0)

    def combine():
        pltpu.sync_copy(hist, shared_big.at[pl.ds(s * (_BINS * _L), _BINS * _L)])
        plsc.subcore_barrier()
        my_words = pl.ds(s * _SLICE * _L, _SLICE * _L)
        pltpu.sync_copy(shared_big.at[pl.ds(s * _SLICE * _L, _SLICE * _L)],
                        redslice)
        for r in range(1, _NSUB):
            pltpu.sync_copy(
                shared_big.at[pl.ds(r * (_BINS * _L) + s * _SLICE * _L,
                                    _SLICE * _L)],
                slicebuf)

            def ab(v, _):
                for u in range(4):
                    o = (v * 4 + u) * _L
                    redslice[pl.ds(o, _L)] = (
                        redslice[pl.ds(o, _L)] + slicebuf[pl.ds(o, _L)])
                return 0
            lax.fori_loop(0, _SLICE // 4, ab, 0)

        for g in range(_SLICE // _L):
            tv = jnp.zeros((_L,), jnp.int32)
            for j in range(_L):
                tot = jnp.sum(redslice[pl.ds((g * _L + j) * _L, _L)])
                tv = jnp.where(lanes == j, tot, tv)
            totstage[pl.ds(g * _L, _L)] = tv
        pltpu.sync_copy(totstage, shared_tot.at[pl.ds(s * _SLICE, _SLICE)])
        plsc.subcore_barrier()
        pltpu.sync_copy(shared_tot, totbuf)

    def scan_for_crossing(start_count):
        def sb(i, carry):
            running, found, bsel, below = carry
            v = totbuf[pl.ds(i * _L, _L)]
            ssum = jnp.sum(v)
            cum = plsc.cumsum(v)
            within = (running + cum) >= k
            lane_pos = jnp.sum(jnp.where(within, 0, 1))
            below_here = running + jnp.sum(jnp.where(lanes < lane_pos, v, 0))
            crossed = jnp.logical_and(running + ssum >= k, found == 0)
            bsel = jnp.where(crossed, i * _L + lane_pos, bsel)
            below = jnp.where(crossed, below_here, below)
            found = jnp.where(crossed, jnp.int32(1), found)
            return running + ssum, found, bsel, below
        init = (start_count, jnp.int32(0), jnp.int32(0), jnp.int32(0))
        _, _, bsel, below = lax.fori_loop(0, _BINS // _L, sb, init)
        return bsel, below

    zero_hist()
    base = s * per_sub
    nch = per_sub // _CH
    cp_o = pltpu.make_async_copy(o_hbm.at[pl.ds(base, per_sub)], e_buf, sem_o)
    cp_o.start()
    tbufs = (tb0, tb1)
    tsems = (sem_t0, sem_t1)
    cps = [None] * nch
    cps[0] = pltpu.make_async_copy(t_hbm.at[pl.ds(base, _CH)], tb0, sem_t0)
    cps[0].start()
    for ch in range(nch):
        if ch + 1 < nch:
            cps[ch + 1] = pltpu.make_async_copy(
                t_hbm.at[pl.ds(base + (ch + 1) * _CH, _CH)],
                tbufs[(ch + 1) % 2], tsems[(ch + 1) % 2])
            cps[ch + 1].start()
        if ch == 0:
            cp_o.wait()
        cps[ch].wait()
        tbuf = tbufs[ch % 2]

        def pa(i, _, ch=ch, tbuf=tbuf):
            for u in range(_U):
                off_v = (i * _U + u) * _L
                o = e_buf[pl.ds(ch * _CH + off_v, _L)]
                t = tbuf[pl.ds(off_v, _L)]
                r = (t - o) / t
                e = r * r
                e_buf[pl.ds(ch * _CH + off_v, _L)] = e
                bits = lax.bitcast_convert_type(e, jnp.int32)
                b1 = lax.shift_right_logical(bits, 21)
                plsc.addupdate_scatter(hist, [b1 * _L + lanes], ones)
            return 0
        lax.fori_loop(0, _CH // (_L * _U), pa, 0)

    combine()
    b1_sel, below1 = scan_for_crossing(jnp.int32(0))

    zero_hist()

    def pb(i, _):
        for u in range(_U):
            off_v = (i * _U + u) * _L
            e = e_buf[pl.ds(off_v, _L)]
            bits = lax.bitcast_convert_type(e, jnp.int32)
            eq = lax.shift_right_logical(bits, 21) == b1_sel
            b2 = lax.bitwise_and(lax.shift_right_logical(bits, 11),
                                 jnp.int32(_BINS - 1))
            plsc.addupdate_scatter(hist, [b2 * _L + lanes], ones, mask=eq)
        return 0
    lax.fori_loop(0, per_sub // (_L * _U), pb, 0)

    combine()
    b2_sel, below2 = scan_for_crossing(below1)

    t_bits = lax.shift_left(b1_sel, 21) | lax.shift_left(b2_sel, 11)

    def pc(i, accs):
        a0, a1 = accs
        for u in range(_U):
            off_v = (i * _U + u) * _L
            e = e_buf[pl.ds(off_v, _L)]
            bits = lax.bitcast_convert_type(e, jnp.int32)
            contrib = jnp.where(bits < t_bits, e, jnp.float32(0.0))
            if u % 2 == 0:
                a0 = a0 + contrib
            else:
                a1 = a1 + contrib
        return a0, a1
    z = jnp.zeros((_L,), jnp.float32)
    a0, a1 = lax.fori_loop(0, per_sub // (_L * _U), pc, (z, z))
    stage_f[...] = a0 + a1
    pltpu.sync_copy(stage_f, shared_f.at[pl.ds(s * _L, _L)])
    plsc.subcore_barrier()

    @pl.when(jnp.logical_and(c == 0, s == 0))
    def _():
        pltpu.sync_copy(shared_f, fbuf)
        acc = fbuf[pl.ds(0, _L)]
        for r in range(1, _NSUB):
            acc = acc + fbuf[pl.ds(r * _L, _L)]
        s_total = jnp.sum(acc)
        t_val = lax.bitcast_convert_type(t_bits, jnp.float32)
        ans = ((s_total + (k - below2).astype(jnp.float32) * t_val)
               * jnp.float32(1.0 / k))
        stage_f[...] = jnp.broadcast_to(ans, (_L,))
        pltpu.sync_copy(stage_f, out_hbm)


def kernel(output, target):
    n = output.shape[0]
    mesh = plsc.VectorSubcoreMesh(
        core_axis_name="c", subcore_axis_name="s",
        num_cores=_NCORE, num_subcores=_NSUB)
    call = pl.kernel(
        functools.partial(_sc_body, n),
        out_type=jax.ShapeDtypeStruct((_L,), jnp.float32),
        mesh=mesh,
        compiler_params=pltpu.CompilerParams(needs_layout_passes=False),
        scratch_types=[
            pltpu.VMEM((_CH,), jnp.float32),
            pltpu.VMEM((_CH,), jnp.float32),
            pltpu.VMEM((n // _NSUB,), jnp.float32),
            pltpu.VMEM((_BINS * _L,), jnp.int32),
            pltpu.VMEM((_SLICE * _L,), jnp.int32),
            pltpu.VMEM((_SLICE * _L,), jnp.int32),
            pltpu.VMEM((_SLICE,), jnp.int32),
            pltpu.VMEM((_BINS,), jnp.int32),
            pltpu.VMEM((_L,), jnp.float32),
            pltpu.VMEM((_NSUB * _L,), jnp.float32),
            pltpu.VMEM_SHARED((_NSUB * _BINS * _L,), jnp.int32),
            pltpu.VMEM_SHARED((_BINS,), jnp.int32),
            pltpu.VMEM_SHARED((_NSUB * _L,), jnp.float32),
            pltpu.SemaphoreType.DMA,
            pltpu.SemaphoreType.DMA,
            pltpu.SemaphoreType.DMA,
        ],
    )
    res = call(output, target)
    return res[0]


# speedup vs baseline: 3.3611x; 2.5325x over previous
import functools

import jax
import jax.numpy as jnp
from jax import lax
from jax.experimental import pallas as pl
from jax.experimental.pallas import tpu as pltpu
from jax.experimental.pallas import tpu_sc as plsc

_L = 16
_NSUB = 16
_NCORE = 2
_BINS = 1024
_CH = 8192
_U = 8
_SLICE = _BINS // _NSUB


def _sc_body(n, o_hbm, t_hbm, out_hbm,
             tb0, tb1, e_buf, hist, slicebuf, redslice, totstage, totbuf,
             stage_f, fbuf, shared_big, shared_tot, shared_f,
             sem_o, sem_t0, sem_t1):
    c = lax.axis_index("c")
    s = lax.axis_index("s")
    per_sub = n // _NSUB
    k = int(n * 0.97)
    lanes = lax.iota(jnp.int32, _L)
    ones = jnp.ones((_L,), jnp.int32)
    zv = jnp.zeros((_L,), jnp.int32)

    def zero_hist():
        @plsc.parallel_loop(0, _BINS * _L, _L, unroll=_U)
        def _zb(off):
            hist[pl.ds(off, _L)] = zv

    def combine():
        pltpu.sync_copy(hist, shared_big.at[pl.ds(s * (_BINS * _L), _BINS * _L)])
        plsc.subcore_barrier()
        my_words = pl.ds(s * _SLICE * _L, _SLICE * _L)
        pltpu.sync_copy(shared_big.at[pl.ds(s * _SLICE * _L, _SLICE * _L)],
                        redslice)
        for r in range(1, _NSUB):
            pltpu.sync_copy(
                shared_big.at[pl.ds(r * (_BINS * _L) + s * _SLICE * _L,
                                    _SLICE * _L)],
                slicebuf)

            @plsc.parallel_loop(0, _SLICE * _L, _L, unroll=4)
            def _ab(o):
                redslice[pl.ds(o, _L)] = (
                    redslice[pl.ds(o, _L)] + slicebuf[pl.ds(o, _L)])

        for g in range(_SLICE // _L):
            tv = jnp.zeros((_L,), jnp.int32)
            for j in range(_L):
                tot = jnp.sum(redslice[pl.ds((g * _L + j) * _L, _L)])
                tv = jnp.where(lanes == j, tot, tv)
            totstage[pl.ds(g * _L, _L)] = tv
        pltpu.sync_copy(totstage, shared_tot.at[pl.ds(s * _SLICE, _SLICE)])
        plsc.subcore_barrier()
        pltpu.sync_copy(shared_tot, totbuf)

    def scan_for_crossing(start_count):
        def sb(i, carry):
            running, found, bsel, below = carry
            v = totbuf[pl.ds(i * _L, _L)]
            ssum = jnp.sum(v)
            cum = plsc.cumsum(v)
            within = (running + cum) >= k
            lane_pos = jnp.sum(jnp.where(within, 0, 1))
            below_here = running + jnp.sum(jnp.where(lanes < lane_pos, v, 0))
            crossed = jnp.logical_and(running + ssum >= k, found == 0)
            bsel = jnp.where(crossed, i * _L + lane_pos, bsel)
            below = jnp.where(crossed, below_here, below)
            found = jnp.where(crossed, jnp.int32(1), found)
            return running + ssum, found, bsel, below
        init = (start_count, jnp.int32(0), jnp.int32(0), jnp.int32(0))
        _, _, bsel, below = lax.fori_loop(0, _BINS // _L, sb, init)
        return bsel, below

    zero_hist()
    base = s * per_sub
    nch = per_sub // _CH
    cp_o = pltpu.make_async_copy(o_hbm.at[pl.ds(base, per_sub)], e_buf, sem_o)
    cp_o.start()
    tbufs = (tb0, tb1)
    tsems = (sem_t0, sem_t1)
    cps = [None] * nch
    cps[0] = pltpu.make_async_copy(t_hbm.at[pl.ds(base, _CH)], tb0, sem_t0)
    cps[0].start()
    for ch in range(nch):
        if ch + 1 < nch:
            cps[ch + 1] = pltpu.make_async_copy(
                t_hbm.at[pl.ds(base + (ch + 1) * _CH, _CH)],
                tbufs[(ch + 1) % 2], tsems[(ch + 1) % 2])
            cps[ch + 1].start()
        if ch == 0:
            cp_o.wait()
        cps[ch].wait()
        tbuf = tbufs[ch % 2]

        @plsc.parallel_loop(0, _CH, _L, unroll=_U)
        def _pa(off, ch=ch, tbuf=tbuf):
            o = e_buf[pl.ds(ch * _CH + off, _L)]
            t = tbuf[pl.ds(off, _L)]
            r = (t - o) / t
            e = r * r
            e_buf[pl.ds(ch * _CH + off, _L)] = e
            bits = lax.bitcast_convert_type(e, jnp.int32)
            b1 = lax.shift_right_logical(bits, 21)
            plsc.addupdate_scatter(hist, [b1 * _L + lanes], ones)

    combine()
    b1_sel, below1 = scan_for_crossing(jnp.int32(0))

    zero_hist()

    @plsc.parallel_loop(0, per_sub, _L, unroll=_U)
    def _pb(off):
        e = e_buf[pl.ds(off, _L)]
        bits = lax.bitcast_convert_type(e, jnp.int32)
        eq = lax.shift_right_logical(bits, 21) == b1_sel
        b2 = lax.bitwise_and(lax.shift_right_logical(bits, 11),
                             jnp.int32(_BINS - 1))
        plsc.addupdate_scatter(hist, [b2 * _L + lanes], ones, mask=eq)

    combine()
    b2_sel, below2 = scan_for_crossing(below1)

    t_bits = lax.shift_left(b1_sel, 21) | lax.shift_left(b2_sel, 11)

    def pc(i, accs):
        a0, a1 = accs
        for u in range(_U):
            off_v = (i * _U + u) * _L
            e = e_buf[pl.ds(off_v, _L)]
            bits = lax.bitcast_convert_type(e, jnp.int32)
            contrib = jnp.where(bits < t_bits, e, jnp.float32(0.0))
            if u % 2 == 0:
                a0 = a0 + contrib
            else:
                a1 = a1 + contrib
        return a0, a1
    z = jnp.zeros((_L,), jnp.float32)
    a0, a1 = lax.fori_loop(0, per_sub // (_L * _U), pc, (z, z))
    stage_f[...] = a0 + a1
    pltpu.sync_copy(stage_f, shared_f.at[pl.ds(s * _L, _L)])
    plsc.subcore_barrier()

    @pl.when(jnp.logical_and(c == 0, s == 0))
    def _():
        pltpu.sync_copy(shared_f, fbuf)
        acc = fbuf[pl.ds(0, _L)]
        for r in range(1, _NSUB):
            acc = acc + fbuf[pl.ds(r * _L, _L)]
        s_total = jnp.sum(acc)
        t_val = lax.bitcast_convert_type(t_bits, jnp.float32)
        ans = ((s_total + (k - below2).astype(jnp.float32) * t_val)
               * jnp.float32(1.0 / k))
        stage_f[...] = jnp.broadcast_to(ans, (_L,))
        pltpu.sync_copy(stage_f, out_hbm)


def kernel(output, target):
    n = output.shape[0]
    mesh = plsc.VectorSubcoreMesh(
        core_axis_name="c", subcore_axis_name="s",
        num_cores=_NCORE, num_subcores=_NSUB)
    call = pl.kernel(
        functools.partial(_sc_body, n),
        out_type=jax.ShapeDtypeStruct((_L,), jnp.float32),
        mesh=mesh,
        compiler_params=pltpu.CompilerParams(needs_layout_passes=False),
        scratch_types=[
            pltpu.VMEM((_CH,), jnp.float32),
            pltpu.VMEM((_CH,), jnp.float32),
            pltpu.VMEM((n // _NSUB,), jnp.float32),
            pltpu.VMEM((_BINS * _L,), jnp.int32),
            pltpu.VMEM((_SLICE * _L,), jnp.int32),
            pltpu.VMEM((_SLICE * _L,), jnp.int32),
            pltpu.VMEM((_SLICE,), jnp.int32),
            pltpu.VMEM((_BINS,), jnp.int32),
            pltpu.VMEM((_L,), jnp.float32),
            pltpu.VMEM((_NSUB * _L,), jnp.float32),
            pltpu.VMEM_SHARED((_NSUB * _BINS * _L,), jnp.int32),
            pltpu.VMEM_SHARED((_BINS,), jnp.int32),
            pltpu.VMEM_SHARED((_NSUB * _L,), jnp.float32),
            pltpu.SemaphoreType.DMA,
            pltpu.SemaphoreType.DMA,
            pltpu.SemaphoreType.DMA,
        ],
    )
    res = call(output, target)
    return res[0]
